# baseline (speedup 1.0000x reference)
import functools

import jax
import jax.numpy as jnp
from jax.experimental import pallas as pl
from jax.experimental.pallas import tpu as pltpu

_VMEM_LIMIT = 60 * 1024 * 1024


def _prep_body(adj_ref, x_ref, w0_ref, g_ref, d_ref, t0_ref):
    a = adj_ref[...]
    deg = jnp.sum(a, axis=1, keepdims=True) + 1.0
    deg = jnp.maximum(deg, 1.0)
    d = jax.lax.rsqrt(deg)
    d_ref[...] = d

    g_ref[...] = a.astype(g_ref.dtype)

    p0 = (d * x_ref[...]).astype(jnp.bfloat16)
    t0_ref[...] = jnp.dot(
        p0, w0_ref[...], preferred_element_type=jnp.float32
    ).astype(jnp.bfloat16)


def _mid_layer_body(g_ref, t_ref, d_ref, w_ref, o_ref, *, tm):
    i = pl.program_id(0)
    g = g_ref[...].astype(jnp.bfloat16)
    r = jnp.dot(g, t_ref[...], preferred_element_type=jnp.float32)
    r = r + t_ref[pl.ds(i * tm, tm), :].astype(jnp.float32)
    r = jnp.maximum(r, 0.0)
    d = d_ref[...]
    p = (r * (d * d)).astype(jnp.bfloat16)
    o_ref[...] = jnp.dot(
        p, w_ref[...], preferred_element_type=jnp.float32
    ).astype(jnp.bfloat16)


def _last_layer_body(g_ref, t_ref, d_ref, o_ref, *, tm):
    i = pl.program_id(0)
    g = g_ref[...].astype(jnp.bfloat16)
    r = jnp.dot(g, t_ref[...], preferred_element_type=jnp.float32)
    r = r + t_ref[pl.ds(i * tm, tm), :].astype(jnp.float32)
    o_ref[...] = r * d_ref[...]


def _compiler_params():
    return pltpu.CompilerParams(
        dimension_semantics=("arbitrary",),
        vmem_limit_bytes=_VMEM_LIMIT,
    )


def kernel(adj, features, w0, w1, w2):
    n = adj.shape[0]
    f_in = features.shape[1]
    f_h1 = w0.shape[1]
    f_h2 = w1.shape[1]
    f_out = w2.shape[1]

    w0b = w0.astype(jnp.bfloat16)
    w1b = w1.astype(jnp.bfloat16)
    w2b = w2.astype(jnp.bfloat16)

    tm_p = min(1024, n)
    g_mat, d_vec, t0 = pl.pallas_call(
        _prep_body,
        grid=(n // tm_p,),
        in_specs=[
            pl.BlockSpec((tm_p, n), lambda i: (i, 0)),
            pl.BlockSpec((tm_p, f_in), lambda i: (i, 0)),
            pl.BlockSpec((f_in, f_h1), lambda i: (0, 0)),
        ],
        out_specs=[
            pl.BlockSpec((tm_p, n), lambda i: (i, 0)),
            pl.BlockSpec((tm_p, 1), lambda i: (i, 0)),
            pl.BlockSpec((tm_p, f_h1), lambda i: (i, 0)),
        ],
        out_shape=[
            jax.ShapeDtypeStruct((n, n), jnp.float8_e4m3fn),
            jax.ShapeDtypeStruct((n, 1), jnp.float32),
            jax.ShapeDtypeStruct((n, f_h1), jnp.bfloat16),
        ],
        compiler_params=_compiler_params(),
    )(adj, features, w0b)

    tm = min(2048, n)
    grid = (n // tm,)

    def mid_layer(t, w, f_from, f_to):
        return pl.pallas_call(
            functools.partial(_mid_layer_body, tm=tm),
            grid=grid,
            in_specs=[
                pl.BlockSpec((tm, n), lambda i: (i, 0)),
                pl.BlockSpec((n, f_from), lambda i: (0, 0)),
                pl.BlockSpec((tm, 1), lambda i: (i, 0)),
                pl.BlockSpec((f_from, f_to), lambda i: (0, 0)),
            ],
            out_specs=pl.BlockSpec((tm, f_to), lambda i: (i, 0)),
            out_shape=jax.ShapeDtypeStruct((n, f_to), jnp.bfloat16),
            compiler_params=_compiler_params(),
        )(g_mat, t, d_vec, w)

    t1 = mid_layer(t0, w1b, f_h1, f_h2)
    t2 = mid_layer(t1, w2b, f_h2, f_out)

    out = pl.pallas_call(
        functools.partial(_last_layer_body, tm=tm),
        grid=grid,
        in_specs=[
            pl.BlockSpec((tm, n), lambda i: (i, 0)),
            pl.BlockSpec((n, f_out), lambda i: (0, 0)),
            pl.BlockSpec((tm, 1), lambda i: (i, 0)),
        ],
        out_specs=pl.BlockSpec((tm, f_out), lambda i: (i, 0)),
        out_shape=jax.ShapeDtypeStruct((n, f_out), jnp.float32),
        compiler_params=_compiler_params(),
    )(g_mat, t2, d_vec)

    return out


# speedup vs baseline: 1.0063x; 1.0063x over previous
import functools

import jax
import jax.numpy as jnp
from jax.experimental import pallas as pl
from jax.experimental.pallas import tpu as pltpu

_VMEM_LIMIT = 60 * 1024 * 1024


def _prep_body(adj_ref, x_ref, w0_ref, g_ref, d_ref, t0_ref):
    a = adj_ref[...]
    deg = jnp.sum(a, axis=1, keepdims=True) + 1.0
    deg = jnp.maximum(deg, 1.0)
    d = jax.lax.rsqrt(deg)
    d_ref[...] = d

    g_ref[...] = a.astype(g_ref.dtype)

    p0 = (d * x_ref[...]).astype(jnp.bfloat16)
    t0_ref[...] = jnp.dot(
        p0, w0_ref[...], preferred_element_type=jnp.float32
    ).astype(jnp.bfloat16)


def _mid_layer_body(g_ref, t_ref, d_ref, w_ref, o_ref, *, tm):
    i = pl.program_id(0)
    g = g_ref[...].astype(jnp.bfloat16)
    r = jnp.dot(g, t_ref[...], preferred_element_type=jnp.float32)
    r = r + t_ref[pl.ds(i * tm, tm), :].astype(jnp.float32)
    r = jnp.maximum(r, 0.0)
    d = d_ref[...]
    p = (r * (d * d)).astype(jnp.bfloat16)
    o_ref[...] = jnp.dot(
        p, w_ref[...], preferred_element_type=jnp.float32
    ).astype(jnp.bfloat16)


def _last_layer_body(g_ref, t_ref, d_ref, o_ref, *, tm):
    i = pl.program_id(0)
    g = g_ref[...].astype(jnp.bfloat16)
    r = jnp.dot(g, t_ref[...], preferred_element_type=jnp.float32)
    r = r + t_ref[pl.ds(i * tm, tm), :].astype(jnp.float32)
    o_ref[...] = r * d_ref[...]


def _compiler_params():
    return pltpu.CompilerParams(
        dimension_semantics=("arbitrary",),
        vmem_limit_bytes=_VMEM_LIMIT,
    )


def kernel(adj, features, w0, w1, w2):
    n = adj.shape[0]
    f_in = features.shape[1]
    f_h1 = w0.shape[1]
    f_h2 = w1.shape[1]
    f_out = w2.shape[1]

    w0b = w0.astype(jnp.bfloat16)
    w1b = w1.astype(jnp.bfloat16)
    w2b = w2.astype(jnp.bfloat16)

    tm_p = min(512, n)
    g_mat, d_vec, t0 = pl.pallas_call(
        _prep_body,
        grid=(n // tm_p,),
        in_specs=[
            pl.BlockSpec((tm_p, n), lambda i: (i, 0)),
            pl.BlockSpec((tm_p, f_in), lambda i: (i, 0)),
            pl.BlockSpec((f_in, f_h1), lambda i: (0, 0)),
        ],
        out_specs=[
            pl.BlockSpec((tm_p, n), lambda i: (i, 0)),
            pl.BlockSpec((tm_p, 1), lambda i: (i, 0)),
            pl.BlockSpec((tm_p, f_h1), lambda i: (i, 0)),
        ],
        out_shape=[
            jax.ShapeDtypeStruct((n, n), jnp.float8_e4m3fn),
            jax.ShapeDtypeStruct((n, 1), jnp.float32),
            jax.ShapeDtypeStruct((n, f_h1), jnp.bfloat16),
        ],
        compiler_params=_compiler_params(),
    )(adj, features, w0b)

    tm = min(2048, n)
    grid = (n // tm,)

    def mid_layer(t, w, f_from, f_to):
        return pl.pallas_call(
            functools.partial(_mid_layer_body, tm=tm),
            grid=grid,
            in_specs=[
                pl.BlockSpec((tm, n), lambda i: (i, 0)),
                pl.BlockSpec((n, f_from), lambda i: (0, 0)),
                pl.BlockSpec((tm, 1), lambda i: (i, 0)),
                pl.BlockSpec((f_from, f_to), lambda i: (0, 0)),
            ],
            out_specs=pl.BlockSpec((tm, f_to), lambda i: (i, 0)),
            out_shape=jax.ShapeDtypeStruct((n, f_to), jnp.bfloat16),
            compiler_params=_compiler_params(),
        )(g_mat, t, d_vec, w)

    t1 = mid_layer(t0, w1b, f_h1, f_h2)
    t2 = mid_layer(t1, w2b, f_h2, f_out)

    out = pl.pallas_call(
        functools.partial(_last_layer_body, tm=tm),
        grid=grid,
        in_specs=[
            pl.BlockSpec((tm, n), lambda i: (i, 0)),
            pl.BlockSpec((n, f_out), lambda i: (0, 0)),
            pl.BlockSpec((tm, 1), lambda i: (i, 0)),
        ],
        out_specs=pl.BlockSpec((tm, f_out), lambda i: (i, 0)),
        out_shape=jax.ShapeDtypeStruct((n, f_out), jnp.float32),
        compiler_params=_compiler_params(),
    )(g_mat, t2, d_vec)

    return out


# speedup vs baseline: 1.0770x; 1.0703x over previous
import functools

import jax
import jax.numpy as jnp
from jax.experimental import pallas as pl
from jax.experimental.pallas import tpu as pltpu

_VMEM_LIMIT = 60 * 1024 * 1024


def _prep_body(adj_ref, x_ref, w0_ref, g_ref, d_ref, t0_ref):
    a = adj_ref[...]
    deg = jnp.sum(a, axis=1, keepdims=True) + 1.0
    deg = jnp.maximum(deg, 1.0)
    d = jax.lax.rsqrt(deg)
    d_ref[...] = d

    g_ref[...] = a.astype(g_ref.dtype)

    p0 = (d * x_ref[...]).astype(jnp.bfloat16)
    t0_ref[...] = jnp.dot(
        p0, w0_ref[...], preferred_element_type=jnp.float32
    ).astype(jnp.bfloat16)


def _mid_layer_body(g_ref, t_ref, d_ref, w_ref, o_ref, *, tm):
    i = pl.program_id(0)
    g = g_ref[...].astype(jnp.bfloat16)
    r = jnp.dot(g, t_ref[...], preferred_element_type=jnp.float32)
    r = r + t_ref[pl.ds(i * tm, tm), :].astype(jnp.float32)
    r = jnp.maximum(r, 0.0)
    d = d_ref[...]
    p = (r * (d * d)).astype(jnp.bfloat16)
    o_ref[...] = jnp.dot(
        p, w_ref[...], preferred_element_type=jnp.float32
    ).astype(jnp.bfloat16)


def _last_layer_body(g_ref, t_ref, d_ref, o_ref, *, tm):
    i = pl.program_id(0)
    g = g_ref[...].astype(jnp.bfloat16)
    r = jnp.dot(g, t_ref[...], preferred_element_type=jnp.float32)
    r = r + t_ref[pl.ds(i * tm, tm), :].astype(jnp.float32)
    o_ref[...] = r * d_ref[...]


def _compiler_params():
    return pltpu.CompilerParams(
        dimension_semantics=("arbitrary",),
        vmem_limit_bytes=_VMEM_LIMIT,
    )


def kernel(adj, features, w0, w1, w2):
    n = adj.shape[0]
    f_in = features.shape[1]
    f_h1 = w0.shape[1]
    f_h2 = w1.shape[1]
    f_out = w2.shape[1]

    w0b = w0.astype(jnp.bfloat16)
    w1b = w1.astype(jnp.bfloat16)
    w2b = w2.astype(jnp.bfloat16)

    tm_p = min(256, n)
    g_mat, d_vec, t0 = pl.pallas_call(
        _prep_body,
        grid=(n // tm_p,),
        in_specs=[
            pl.BlockSpec((tm_p, n), lambda i: (i, 0)),
            pl.BlockSpec((tm_p, f_in), lambda i: (i, 0)),
            pl.BlockSpec((f_in, f_h1), lambda i: (0, 0)),
        ],
        out_specs=[
            pl.BlockSpec((tm_p, n), lambda i: (i, 0)),
            pl.BlockSpec((tm_p, 1), lambda i: (i, 0)),
            pl.BlockSpec((tm_p, f_h1), lambda i: (i, 0)),
        ],
        out_shape=[
            jax.ShapeDtypeStruct((n, n), jnp.float8_e4m3fn),
            jax.ShapeDtypeStruct((n, 1), jnp.float32),
            jax.ShapeDtypeStruct((n, f_h1), jnp.bfloat16),
        ],
        compiler_params=_compiler_params(),
    )(adj, features, w0b)

    tm = min(1024, n)
    grid = (n // tm,)

    def mid_layer(t, w, f_from, f_to):
        return pl.pallas_call(
            functools.partial(_mid_layer_body, tm=tm),
            grid=grid,
            in_specs=[
                pl.BlockSpec((tm, n), lambda i: (i, 0)),
                pl.BlockSpec((n, f_from), lambda i: (0, 0)),
                pl.BlockSpec((tm, 1), lambda i: (i, 0)),
                pl.BlockSpec((f_from, f_to), lambda i: (0, 0)),
            ],
            out_specs=pl.BlockSpec((tm, f_to), lambda i: (i, 0)),
            out_shape=jax.ShapeDtypeStruct((n, f_to), jnp.bfloat16),
            compiler_params=_compiler_params(),
        )(g_mat, t, d_vec, w)

    t1 = mid_layer(t0, w1b, f_h1, f_h2)
    t2 = mid_layer(t1, w2b, f_h2, f_out)

    out = pl.pallas_call(
        functools.partial(_last_layer_body, tm=tm),
        grid=grid,
        in_specs=[
            pl.BlockSpec((tm, n), lambda i: (i, 0)),
            pl.BlockSpec((n, f_out), lambda i: (0, 0)),
            pl.BlockSpec((tm, 1), lambda i: (i, 0)),
        ],
        out_specs=pl.BlockSpec((tm, f_out), lambda i: (i, 0)),
        out_shape=jax.ShapeDtypeStruct((n, f_out), jnp.float32),
        compiler_params=_compiler_params(),
    )(g_mat, t2, d_vec)

    return out


# speedup vs baseline: 1.1850x; 1.1003x over previous
import functools

import jax
import jax.numpy as jnp
from jax.experimental import pallas as pl
from jax.experimental.pallas import tpu as pltpu

_VMEM_LIMIT = 60 * 1024 * 1024


def _prep_body(adj_ref, x_ref, w0_ref, g_ref, d_ref, t0_ref):
    a = adj_ref[...]
    deg = jnp.sum(a, axis=1, keepdims=True) + 1.0
    deg = jnp.maximum(deg, 1.0)
    d = jax.lax.rsqrt(deg)
    d_ref[...] = d

    g_ref[...] = a.astype(g_ref.dtype)

    p0 = (d * x_ref[...]).astype(jnp.bfloat16)
    t0_ref[...] = jnp.dot(
        p0, w0_ref[...], preferred_element_type=jnp.float32
    ).astype(jnp.bfloat16)


def _mid_layer_body(g_ref, t_ref, d_ref, w_ref, o_ref, *, tm):
    i = pl.program_id(0)
    g = g_ref[...].astype(jnp.bfloat16)
    r = jnp.dot(g, t_ref[...], preferred_element_type=jnp.float32)
    r = r + t_ref[pl.ds(i * tm, tm), :].astype(jnp.float32)
    r = jnp.maximum(r, 0.0)
    d = d_ref[...]
    p = (r * (d * d)).astype(jnp.bfloat16)
    o_ref[...] = jnp.dot(
        p, w_ref[...], preferred_element_type=jnp.float32
    ).astype(jnp.bfloat16)


def _last_layer_body(g_ref, t_ref, d_ref, o_ref, *, tm):
    i = pl.program_id(0)
    g = g_ref[...].astype(jnp.bfloat16)
    r = jnp.dot(g, t_ref[...], preferred_element_type=jnp.float32)
    r = r + t_ref[pl.ds(i * tm, tm), :].astype(jnp.float32)
    o_ref[...] = r * d_ref[...]


def _compiler_params():
    return pltpu.CompilerParams(
        dimension_semantics=("arbitrary",),
        vmem_limit_bytes=_VMEM_LIMIT,
    )


def kernel(adj, features, w0, w1, w2):
    n = adj.shape[0]
    f_in = features.shape[1]
    f_h1 = w0.shape[1]
    f_h2 = w1.shape[1]
    f_out = w2.shape[1]

    w0b = w0.astype(jnp.bfloat16)
    w1b = w1.astype(jnp.bfloat16)
    w2b = w2.astype(jnp.bfloat16)

    tm_p = min(512, n)
    g_mat, d_vec, t0 = pl.pallas_call(
        _prep_body,
        grid=(n // tm_p,),
        in_specs=[
            pl.BlockSpec((tm_p, n), lambda i: (i, 0)),
            pl.BlockSpec((tm_p, f_in), lambda i: (i, 0)),
            pl.BlockSpec((f_in, f_h1), lambda i: (0, 0)),
        ],
        out_specs=[
            pl.BlockSpec((tm_p, n), lambda i: (i, 0)),
            pl.BlockSpec((tm_p, 1), lambda i: (i, 0)),
            pl.BlockSpec((tm_p, f_h1), lambda i: (i, 0)),
        ],
        out_shape=[
            jax.ShapeDtypeStruct((n, n), jnp.uint4),
            jax.ShapeDtypeStruct((n, 1), jnp.float32),
            jax.ShapeDtypeStruct((n, f_h1), jnp.bfloat16),
        ],
        compiler_params=_compiler_params(),
    )(adj, features, w0b)

    tm = min(1024, n)
    grid = (n // tm,)

    def mid_layer(t, w, f_from, f_to):
        return pl.pallas_call(
            functools.partial(_mid_layer_body, tm=tm),
            grid=grid,
            in_specs=[
                pl.BlockSpec((tm, n), lambda i: (i, 0)),
                pl.BlockSpec((n, f_from), lambda i: (0, 0)),
                pl.BlockSpec((tm, 1), lambda i: (i, 0)),
                pl.BlockSpec((f_from, f_to), lambda i: (0, 0)),
            ],
            out_specs=pl.BlockSpec((tm, f_to), lambda i: (i, 0)),
            out_shape=jax.ShapeDtypeStruct((n, f_to), jnp.bfloat16),
            compiler_params=_compiler_params(),
        )(g_mat, t, d_vec, w)

    t1 = mid_layer(t0, w1b, f_h1, f_h2)
    t2 = mid_layer(t1, w2b, f_h2, f_out)

    out = pl.pallas_call(
        functools.partial(_last_layer_body, tm=tm),
        grid=grid,
        in_specs=[
            pl.BlockSpec((tm, n), lambda i: (i, 0)),
            pl.BlockSpec((n, f_out), lambda i: (0, 0)),
            pl.BlockSpec((tm, 1), lambda i: (i, 0)),
        ],
        out_specs=pl.BlockSpec((tm, f_out), lambda i: (i, 0)),
        out_shape=jax.ShapeDtypeStruct((n, f_out), jnp.float32),
        compiler_params=_compiler_params(),
    )(g_mat, t2, d_vec)

    return out


# speedup vs baseline: 1.2094x; 1.0206x over previous
import functools

import jax
import jax.numpy as jnp
from jax.experimental import pallas as pl
from jax.experimental.pallas import tpu as pltpu

_VMEM_LIMIT = 60 * 1024 * 1024


def _prep_body(adj_ref, x_ref, w0_ref, g_ref, d_ref, t0_ref):
    a = adj_ref[...]
    deg = jnp.sum(a, axis=1, keepdims=True) + 1.0
    deg = jnp.maximum(deg, 1.0)
    d = jax.lax.rsqrt(deg)
    d_ref[...] = d

    g_ref[...] = a.astype(g_ref.dtype)

    p0 = (d * x_ref[...]).astype(jnp.bfloat16)
    t0_ref[...] = jnp.dot(
        p0, w0_ref[...], preferred_element_type=jnp.float32
    ).astype(jnp.bfloat16)


def _mid_layer_body(g_ref, t_ref, d_ref, w_ref, o_ref, *, tm):
    i = pl.program_id(0)
    g = g_ref[...].astype(jnp.bfloat16)
    r = jnp.dot(g, t_ref[...], preferred_element_type=jnp.float32)
    r = r + t_ref[pl.ds(i * tm, tm), :].astype(jnp.float32)
    r = jnp.maximum(r, 0.0)
    d = d_ref[...]
    p = (r * (d * d)).astype(jnp.bfloat16)
    o_ref[...] = jnp.dot(
        p, w_ref[...], preferred_element_type=jnp.float32
    ).astype(jnp.bfloat16)


def _last_layer_body(g_ref, t_ref, d_ref, o_ref, *, tm):
    i = pl.program_id(0)
    g = g_ref[...].astype(jnp.bfloat16)
    r = jnp.dot(g, t_ref[...], preferred_element_type=jnp.float32)
    r = r + t_ref[pl.ds(i * tm, tm), :].astype(jnp.float32)
    o_ref[...] = r * d_ref[...]


def _compiler_params():
    return pltpu.CompilerParams(
        dimension_semantics=("arbitrary",),
        vmem_limit_bytes=_VMEM_LIMIT,
    )


def kernel(adj, features, w0, w1, w2):
    n = adj.shape[0]
    f_in = features.shape[1]
    f_h1 = w0.shape[1]
    f_h2 = w1.shape[1]
    f_out = w2.shape[1]

    w0b = w0.astype(jnp.bfloat16)
    w1b = w1.astype(jnp.bfloat16)
    w2b = w2.astype(jnp.bfloat16)

    tm_p = min(512, n)
    g_mat, d_vec, t0 = pl.pallas_call(
        _prep_body,
        grid=(n // tm_p,),
        in_specs=[
            pl.BlockSpec((tm_p, n), lambda i: (i, 0)),
            pl.BlockSpec((tm_p, f_in), lambda i: (i, 0)),
            pl.BlockSpec((f_in, f_h1), lambda i: (0, 0)),
        ],
        out_specs=[
            pl.BlockSpec((tm_p, n), lambda i: (i, 0)),
            pl.BlockSpec((tm_p, 1), lambda i: (i, 0)),
            pl.BlockSpec((tm_p, f_h1), lambda i: (i, 0)),
        ],
        out_shape=[
            jax.ShapeDtypeStruct((n, n), jnp.uint2),
            jax.ShapeDtypeStruct((n, 1), jnp.float32),
            jax.ShapeDtypeStruct((n, f_h1), jnp.bfloat16),
        ],
        compiler_params=_compiler_params(),
    )(adj, features, w0b)

    tm = min(1024, n)
    grid = (n // tm,)

    def mid_layer(t, w, f_from, f_to):
        return pl.pallas_call(
            functools.partial(_mid_layer_body, tm=tm),
            grid=grid,
            in_specs=[
                pl.BlockSpec((tm, n), lambda i: (i, 0)),
                pl.BlockSpec((n, f_from), lambda i: (0, 0)),
                pl.BlockSpec((tm, 1), lambda i: (i, 0)),
                pl.BlockSpec((f_from, f_to), lambda i: (0, 0)),
            ],
            out_specs=pl.BlockSpec((tm, f_to), lambda i: (i, 0)),
            out_shape=jax.ShapeDtypeStruct((n, f_to), jnp.bfloat16),
            compiler_params=_compiler_params(),
        )(g_mat, t, d_vec, w)

    t1 = mid_layer(t0, w1b, f_h1, f_h2)
    t2 = mid_layer(t1, w2b, f_h2, f_out)

    out = pl.pallas_call(
        functools.partial(_last_layer_body, tm=tm),
        grid=grid,
        in_specs=[
            pl.BlockSpec((tm, n), lambda i: (i, 0)),
            pl.BlockSpec((n, f_out), lambda i: (0, 0)),
            pl.BlockSpec((tm, 1), lambda i: (i, 0)),
        ],
        out_specs=pl.BlockSpec((tm, f_out), lambda i: (i, 0)),
        out_shape=jax.ShapeDtypeStruct((n, f_out), jnp.float32),
        compiler_params=_compiler_params(),
    )(g_mat, t2, d_vec)

    return out
